# Initial kernel scaffold; baseline (speedup 1.0000x reference)
import functools

import jax
import jax.numpy as jnp
from jax import lax
from jax.experimental import pallas as pl
from jax.experimental.pallas import tpu as pltpu
from jax.experimental.pallas import tpu_sc as plsc

VOCAB = 1_000_000
EMBED_DIM = 32
BATCH = 16384
HIST = 50

NUM_WORKERS = 32
TOTAL = BATCH * HIST
PER_WORKER = TOTAL // NUM_WORKERS
GATHER = 128
NUM_GATHERS = PER_WORKER // GATHER

_mesh = plsc.VectorSubcoreMesh(core_axis_name="c", subcore_axis_name="s")


@functools.partial(
    pl.kernel,
    mesh=_mesh,
    out_type=jax.ShapeDtypeStruct((TOTAL, EMBED_DIM), jnp.float32),
    scratch_types=[
        pltpu.VMEM((NUM_GATHERS, GATHER), jnp.int32),
        pltpu.VMEM((GATHER, EMBED_DIM), jnp.float32),
        pltpu.SemaphoreType.DMA,
    ],
)
def _embed_gather(x_hbm, table_hbm, out_hbm, idx_v, rows_v, sem):
    wid = lax.axis_index("s") * 2 + lax.axis_index("c")
    base = wid * PER_WORKER
    pltpu.sync_copy(x_hbm.at[wid], idx_v)

    def body(j, carry):
        pltpu.async_copy(table_hbm.at[idx_v.at[j]], rows_v, sem).wait()
        pltpu.sync_copy(rows_v, out_hbm.at[pl.ds(base + j * GATHER, GATHER)])
        return carry

    lax.fori_loop(0, NUM_GATHERS, body, 0)


def kernel(x, table):
    xr = x.reshape(NUM_WORKERS, NUM_GATHERS, GATHER)
    out = _embed_gather(xr, table)
    return out.reshape(BATCH, HIST, EMBED_DIM)


# speedup vs baseline: 1.0220x; 1.0220x over previous
import functools

import jax
import jax.numpy as jnp
from jax import lax
from jax.experimental import pallas as pl
from jax.experimental.pallas import tpu as pltpu
from jax.experimental.pallas import tpu_sc as plsc

VOCAB = 1_000_000
EMBED_DIM = 32
BATCH = 16384
HIST = 50

NUM_WORKERS = 32
TOTAL = BATCH * HIST
PER_WORKER = TOTAL // NUM_WORKERS
GATHER = 128
NUM_GATHERS = PER_WORKER // GATHER

_mesh = plsc.VectorSubcoreMesh(core_axis_name="c", subcore_axis_name="s")


@functools.partial(
    pl.kernel,
    mesh=_mesh,
    out_type=jax.ShapeDtypeStruct((TOTAL, EMBED_DIM), jnp.float32),
    compiler_params=pltpu.CompilerParams(use_tc_tiling_on_sc=False),
    scratch_types=[
        pltpu.VMEM((NUM_GATHERS, GATHER), jnp.int32),
        pltpu.VMEM((GATHER, EMBED_DIM), jnp.float32),
        pltpu.SemaphoreType.DMA,
    ],
)
def _embed_gather(x_hbm, table_hbm, out_hbm, idx_v, rows_v, sem):
    wid = lax.axis_index("s") * 2 + lax.axis_index("c")
    base = wid * PER_WORKER
    pltpu.sync_copy(x_hbm.at[wid], idx_v)

    def body(j, carry):
        pltpu.async_copy(table_hbm.at[idx_v.at[j]], rows_v, sem).wait()
        pltpu.sync_copy(rows_v, out_hbm.at[pl.ds(base + j * GATHER, GATHER)])
        return carry

    lax.fori_loop(0, NUM_GATHERS, body, 0)


def kernel(x, table):
    xr = x.reshape(NUM_WORKERS, NUM_GATHERS, GATHER)
    out = _embed_gather(xr, table)
    return out.reshape(BATCH, HIST, EMBED_DIM)


# speedup vs baseline: 1.1127x; 1.0887x over previous
import functools

import jax
import jax.numpy as jnp
from jax import lax
from jax.experimental import pallas as pl
from jax.experimental.pallas import tpu as pltpu
from jax.experimental.pallas import tpu_sc as plsc

VOCAB = 1_000_000
EMBED_DIM = 32
BATCH = 16384
HIST = 50

NUM_WORKERS = 32
TOTAL = BATCH * HIST
PER_WORKER = TOTAL // NUM_WORKERS
GATHER = 128
NUM_GATHERS = PER_WORKER // GATHER
NBUF = 8
NUM_GROUPS = NUM_GATHERS // NBUF

_mesh = plsc.VectorSubcoreMesh(core_axis_name="c", subcore_axis_name="s")


@functools.partial(
    pl.kernel,
    mesh=_mesh,
    out_type=jax.ShapeDtypeStruct((TOTAL, EMBED_DIM), jnp.float32),
    compiler_params=pltpu.CompilerParams(use_tc_tiling_on_sc=False),
    scratch_types=[
        pltpu.VMEM((NUM_GATHERS, GATHER), jnp.int32),
        pltpu.VMEM((NBUF, GATHER, EMBED_DIM), jnp.float32),
        [pltpu.SemaphoreType.DMA] * NBUF,
    ],
)
def _embed_gather(x_hbm, table_hbm, out_hbm, idx_v, rows_v, semg):
    wid = lax.axis_index("s") * 2 + lax.axis_index("c")
    base = wid * PER_WORKER
    pltpu.sync_copy(x_hbm.at[wid], idx_v)

    def gather_chunk(j, b, start=True):
        mk = pltpu.async_copy if start else pltpu.make_async_copy
        return mk(table_hbm.at[idx_v.at[j]], rows_v.at[b], semg[b])

    for b in range(NBUF):
        gather_chunk(b, b)

    def body(gi, carry):
        for b in range(NBUF):
            j = gi * NBUF + b
            gather_chunk(j, b, start=False).wait()
            pltpu.sync_copy(
                rows_v.at[b], out_hbm.at[pl.ds(base + j * GATHER, GATHER)])

            @pl.when(gi < NUM_GROUPS - 1)
            def _():
                gather_chunk(j + NBUF, b)
        return carry

    lax.fori_loop(0, NUM_GROUPS, body, 0)


def kernel(x, table):
    xr = x.reshape(NUM_WORKERS, NUM_GATHERS, GATHER)
    out = _embed_gather(xr, table)
    return out.reshape(BATCH, HIST, EMBED_DIM)
